# Initial kernel scaffold; baseline (speedup 1.0000x reference)
import jax
import jax.numpy as jnp
from jax.experimental import pallas as pl


def kernel(x, event, W_pre, b_pre, W_post, b_post):
    raise NotImplementedError("write your pallas kernel here")


# speedup vs baseline: 5.3869x; 5.3869x over previous
import jax
import jax.numpy as jnp
from jax.experimental import pallas as pl
from jax.experimental.pallas import tpu as pltpu

N_NODES = 100000
D = 128
E = 1024
R = 2000
NB = N_NODES // R


def _fused_body(ev_ref, x_ref, wpre_ref, bpre_ref, wpost_ref, bpost_ref,
                out_ref, gsum_ref, cnt_ref):
    i = pl.program_id(0)

    @pl.when(i == 0)
    def _init():
        gsum_ref[...] = jnp.zeros_like(gsum_ref)
        cnt_ref[...] = jnp.zeros_like(cnt_ref)

    xb = x_ref[...].astype(jnp.bfloat16)
    wb = wpre_ref[...].astype(jnp.bfloat16)
    pre = jax.lax.dot_general(xb, wb, (((1,), (0,)), ((), ())),
                              preferred_element_type=jnp.float32)
    ftx = jnp.maximum(pre + bpre_ref[...], 0.0)

    ev = ev_ref[0, 0, :]
    iota = jax.lax.broadcasted_iota(jnp.int32, (E, R), 0)
    oh = iota == jnp.broadcast_to(ev[None, :], (E, R))
    ohb = oh.astype(jnp.bfloat16)
    gsum_ref[...] += jax.lax.dot_general(
        ohb, ftx.astype(jnp.bfloat16), (((1,), (0,)), ((), ())),
        preferred_element_type=jnp.float32)
    cnt_ref[...] += jnp.sum(oh.astype(jnp.float32), axis=1, keepdims=True)

    @pl.when(i == NB - 1)
    def _finish():
        gsum = gsum_ref[...]
        gmean = gsum * (1.0 / jnp.maximum(cnt_ref[...], 1.0))
        w1 = wpost_ref[:D, :]
        w2 = wpost_ref[D:, :]
        out_ref[...] = (
            jax.lax.dot_general(gsum, w1, (((1,), (0,)), ((), ())),
                                preferred_element_type=jnp.float32)
            + jax.lax.dot_general(gmean, w2, (((1,), (0,)), ((), ())),
                                  preferred_element_type=jnp.float32)
            + bpost_ref[...])


def kernel(x, event, W_pre, b_pre, W_post, b_post):
    ev3 = event.astype(jnp.int32).reshape(NB, 1, R)
    return pl.pallas_call(
        _fused_body,
        grid=(NB,),
        in_specs=[
            pl.BlockSpec((1, 1, R), lambda i: (i, 0, 0)),
            pl.BlockSpec((R, D), lambda i: (i, 0)),
            pl.BlockSpec((D, D), lambda i: (0, 0)),
            pl.BlockSpec((1, D), lambda i: (0, 0)),
            pl.BlockSpec((2 * D, D), lambda i: (0, 0)),
            pl.BlockSpec((1, D), lambda i: (0, 0)),
        ],
        out_specs=pl.BlockSpec((E, D), lambda i: (0, 0)),
        out_shape=jax.ShapeDtypeStruct((E, D), jnp.float32),
        scratch_shapes=[
            pltpu.VMEM((E, D), jnp.float32),
            pltpu.VMEM((E, 1), jnp.float32),
        ],
    )(ev3, x, W_pre, b_pre.reshape(1, D), W_post, b_post.reshape(1, D))


# speedup vs baseline: 7.5810x; 1.4073x over previous
import jax
import jax.numpy as jnp
from jax.experimental import pallas as pl
from jax.experimental.pallas import tpu as pltpu

N_NODES = 100000
D = 128
E = 1024
R = 2000
NB = N_NODES // R
W = 256
G = 64
EPAD = E + W


def _body(s_ref, ev_ref, x_ref, wpre_ref, bpre_ref, wpost_ref, bpost_ref,
          out_ref, gsum_ref, cnt_ref, ftx_ref):
    i = pl.program_id(0)
    t = s_ref[0, i]
    ws = s_ref[1, i]
    valid = s_ref[2, i] == 1
    prev_t = jnp.where(i == 0, -1, s_ref[0, jnp.maximum(i - 1, 0)])

    @pl.when(i == 0)
    def _init():
        gsum_ref[...] = jnp.zeros_like(gsum_ref)
        cnt_ref[...] = jnp.zeros_like(cnt_ref)
        ftx_ref[:, D:] = jnp.ones((R, D), jnp.bfloat16)

    @pl.when(t != prev_t)
    def _new_tile():
        xb = x_ref[0].astype(jnp.bfloat16)
        wb = wpre_ref[...].astype(jnp.bfloat16)
        pre = jax.lax.dot_general(xb, wb, (((1,), (0,)), ((), ())),
                                  preferred_element_type=jnp.float32)
        ftx_ref[:, :D] = jnp.maximum(pre + bpre_ref[...], 0.0
                                     ).astype(jnp.bfloat16)

    @pl.when(valid)
    def _accum():
        ev_rel = ev_ref[0, 0, :] - ws
        iota = jax.lax.broadcasted_iota(jnp.int32, (W, R), 0)
        ohb = (iota == jnp.broadcast_to(ev_rel[None, :], (W, R))
               ).astype(jnp.bfloat16)
        res = jax.lax.dot_general(ohb, ftx_ref[...], (((1,), (0,)), ((), ())),
                                  preferred_element_type=jnp.float32)
        gsum_ref[pl.ds(ws, W), :] += res[:, :D]
        cnt_ref[pl.ds(ws, W), :] += res[:, D:D + 1]

    @pl.when(i == G - 1)
    def _finish():
        gsum = gsum_ref[:E, :]
        gmean = gsum * (1.0 / jnp.maximum(cnt_ref[:E, :], 1.0))
        w1 = wpost_ref[:D, :]
        w2 = wpost_ref[D:, :]
        out_ref[...] = (
            jax.lax.dot_general(gsum, w1, (((1,), (0,)), ((), ())),
                                preferred_element_type=jnp.float32)
            + jax.lax.dot_general(gmean, w2, (((1,), (0,)), ((), ())),
                                  preferred_element_type=jnp.float32)
            + bpost_ref[...])


def kernel(x, event, W_pre, b_pre, W_post, b_post):
    ev = event.astype(jnp.int32)
    ev2 = ev.reshape(NB, R)
    first = ev2[:, 0]
    last = ev2[:, -1]
    base = first - (first % 8)
    nw = (last - base) // W + 1
    off = jnp.concatenate(
        [jnp.zeros((1,), jnp.int32), jnp.cumsum(nw).astype(jnp.int32)])
    total = off[NB]
    idx = jnp.arange(G, dtype=jnp.int32)
    t = jnp.clip(jnp.searchsorted(off, idx, side='right') - 1, 0, NB - 1
                 ).astype(jnp.int32)
    k = idx - off[t]
    ws = base[t] + k * W
    valid = (idx < total).astype(jnp.int32)
    ws = jnp.where(valid == 1, ws, 0)
    sched = jnp.stack([t, ws, valid]).astype(jnp.int32)

    grid_spec = pltpu.PrefetchScalarGridSpec(
        num_scalar_prefetch=1,
        grid=(G,),
        in_specs=[
            pl.BlockSpec((1, 1, R), lambda i, s: (s[0, i], 0, 0)),
            pl.BlockSpec((1, R, D), lambda i, s: (s[0, i], 0, 0)),
            pl.BlockSpec((D, D), lambda i, s: (0, 0)),
            pl.BlockSpec((1, D), lambda i, s: (0, 0)),
            pl.BlockSpec((2 * D, D), lambda i, s: (0, 0)),
            pl.BlockSpec((1, D), lambda i, s: (0, 0)),
        ],
        out_specs=pl.BlockSpec((E, D), lambda i, s: (0, 0)),
        scratch_shapes=[
            pltpu.VMEM((EPAD, D), jnp.float32),
            pltpu.VMEM((EPAD, 1), jnp.float32),
            pltpu.VMEM((R, 2 * D), jnp.bfloat16),
        ],
    )
    return pl.pallas_call(
        _body,
        grid_spec=grid_spec,
        out_shape=jax.ShapeDtypeStruct((E, D), jnp.float32),
    )(sched, ev.reshape(NB, 1, R), x.reshape(NB, R, D), W_pre,
      b_pre.reshape(1, D), W_post, b_post.reshape(1, D))


# speedup vs baseline: 10.5851x; 1.3963x over previous
import jax
import jax.numpy as jnp
from jax.experimental import pallas as pl
from jax.experimental.pallas import tpu as pltpu

N_NODES = 100000
D = 128
E = 1024
R = 4000
NB = N_NODES // R
W = 256
G = 32
EPAD = E + W


def _body(s_ref, ev_ref, x_ref, wpre_ref, bpre_ref, wpost_ref, bpost_ref,
          out_ref, gsum_ref, cnt_ref, ftx_ref):
    i = pl.program_id(0)
    t = s_ref[0, i]
    ws = s_ref[1, i]
    valid = s_ref[2, i] == 1
    prev_t = jnp.where(i == 0, -1, s_ref[0, jnp.maximum(i - 1, 0)])

    @pl.when(i == 0)
    def _init():
        gsum_ref[...] = jnp.zeros_like(gsum_ref)
        cnt_ref[...] = jnp.zeros_like(cnt_ref)
        ftx_ref[:, D:] = jnp.ones((R, D), jnp.bfloat16)

    @pl.when(t != prev_t)
    def _new_tile():
        xb = x_ref[0].astype(jnp.bfloat16)
        wb = wpre_ref[...].astype(jnp.bfloat16)
        pre = jax.lax.dot_general(xb, wb, (((1,), (0,)), ((), ())),
                                  preferred_element_type=jnp.float32)
        ftx_ref[:, :D] = jnp.maximum(pre + bpre_ref[...], 0.0
                                     ).astype(jnp.bfloat16)

    @pl.when(valid)
    def _accum():
        ev_rel = ev_ref[0, 0, :] - ws
        iota = jax.lax.broadcasted_iota(jnp.int32, (W, R), 0)
        ohb = (iota == jnp.broadcast_to(ev_rel[None, :], (W, R))
               ).astype(jnp.bfloat16)
        h = R // 2
        res = (jax.lax.dot_general(ohb[:, :h], ftx_ref[:h, :],
                                   (((1,), (0,)), ((), ())),
                                   preferred_element_type=jnp.float32)
               + jax.lax.dot_general(ohb[:, h:], ftx_ref[h:, :],
                                     (((1,), (0,)), ((), ())),
                                     preferred_element_type=jnp.float32))
        gsum_ref[pl.ds(ws, W), :] += res[:, :D]
        cnt_ref[pl.ds(ws, W), :] += res[:, D:D + 1]

    @pl.when(i == G - 1)
    def _finish():
        gsum = gsum_ref[:E, :]
        gmean = gsum * (1.0 / jnp.maximum(cnt_ref[:E, :], 1.0))
        w1 = wpost_ref[:D, :]
        w2 = wpost_ref[D:, :]
        out_ref[...] = (
            jax.lax.dot_general(gsum, w1, (((1,), (0,)), ((), ())),
                                preferred_element_type=jnp.float32)
            + jax.lax.dot_general(gmean, w2, (((1,), (0,)), ((), ())),
                                  preferred_element_type=jnp.float32)
            + bpost_ref[...])


def kernel(x, event, W_pre, b_pre, W_post, b_post):
    ev = event.astype(jnp.int32)
    ev2 = ev.reshape(NB, R)
    first = ev2[:, 0]
    last = ev2[:, -1]
    base = first - (first % 8)
    nw = (last - base) // W + 1
    off = jnp.concatenate(
        [jnp.zeros((1,), jnp.int32), jnp.cumsum(nw).astype(jnp.int32)])
    total = off[NB]
    idx = jnp.arange(G, dtype=jnp.int32)
    t = jnp.clip(jnp.searchsorted(off, idx, side='right') - 1, 0, NB - 1
                 ).astype(jnp.int32)
    k = idx - off[t]
    ws = base[t] + k * W
    valid = (idx < total).astype(jnp.int32)
    ws = jnp.where(valid == 1, ws, 0)
    sched = jnp.stack([t, ws, valid]).astype(jnp.int32)

    grid_spec = pltpu.PrefetchScalarGridSpec(
        num_scalar_prefetch=1,
        grid=(G,),
        in_specs=[
            pl.BlockSpec((1, 1, R), lambda i, s: (s[0, i], 0, 0)),
            pl.BlockSpec((1, R, D), lambda i, s: (s[0, i], 0, 0)),
            pl.BlockSpec((D, D), lambda i, s: (0, 0)),
            pl.BlockSpec((1, D), lambda i, s: (0, 0)),
            pl.BlockSpec((2 * D, D), lambda i, s: (0, 0)),
            pl.BlockSpec((1, D), lambda i, s: (0, 0)),
        ],
        out_specs=pl.BlockSpec((E, D), lambda i, s: (0, 0)),
        scratch_shapes=[
            pltpu.VMEM((EPAD, D), jnp.float32),
            pltpu.VMEM((EPAD, 1), jnp.float32),
            pltpu.VMEM((R, 2 * D), jnp.bfloat16),
        ],
    )
    return pl.pallas_call(
        _body,
        grid_spec=grid_spec,
        out_shape=jax.ShapeDtypeStruct((E, D), jnp.float32),
    )(sched, ev.reshape(NB, 1, R), x.reshape(NB, R, D), W_pre,
      b_pre.reshape(1, D), W_post, b_post.reshape(1, D))


# speedup vs baseline: 11.5289x; 1.0892x over previous
import jax
import jax.numpy as jnp
from jax.experimental import pallas as pl
from jax.experimental.pallas import tpu as pltpu

N_NODES = 100000
D = 128
E = 1024
R = 10000
NB = N_NODES // R
W = 256
G = 16
EPAD = E + W


def _body(s_ref, ev_ref, x_ref, wpre_ref, bpre_ref, wpost_ref, bpost_ref,
          out_ref, gsum_ref, cnt_ref, ftx_ref):
    i = pl.program_id(0)
    t = s_ref[0, i]
    ws = s_ref[1, i]
    valid = s_ref[2, i] == 1
    prev_t = jnp.where(i == 0, -1, s_ref[0, jnp.maximum(i - 1, 0)])

    @pl.when(i == 0)
    def _init():
        gsum_ref[...] = jnp.zeros_like(gsum_ref)
        cnt_ref[...] = jnp.zeros_like(cnt_ref)
        ftx_ref[:, D:] = jnp.ones((R, D), jnp.bfloat16)

    @pl.when(t != prev_t)
    def _new_tile():
        xb = x_ref[0].astype(jnp.bfloat16)
        wb = wpre_ref[...].astype(jnp.bfloat16)
        pre = jax.lax.dot_general(xb, wb, (((1,), (0,)), ((), ())),
                                  preferred_element_type=jnp.float32)
        ftx_ref[:, :D] = jnp.maximum(pre + bpre_ref[...], 0.0
                                     ).astype(jnp.bfloat16)

    @pl.when(valid)
    def _accum():
        ev_rel = ev_ref[0, 0, :] - ws
        iota = jax.lax.broadcasted_iota(jnp.int32, (W, R), 0)
        ohb = (iota == jnp.broadcast_to(ev_rel[None, :], (W, R))
               ).astype(jnp.bfloat16)
        h = R // 4
        parts = [jax.lax.dot_general(ohb[:, j * h:(j + 1) * h],
                                     ftx_ref[j * h:(j + 1) * h, :],
                                     (((1,), (0,)), ((), ())),
                                     preferred_element_type=jnp.float32)
                 for j in range(4)]
        res = (parts[0] + parts[1]) + (parts[2] + parts[3])
        gsum_ref[pl.ds(ws, W), :] += res[:, :D]
        cnt_ref[pl.ds(ws, W), :] += res[:, D:D + 1]

    @pl.when(i == G - 1)
    def _finish():
        gsum = gsum_ref[:E, :]
        gmean = gsum * (1.0 / jnp.maximum(cnt_ref[:E, :], 1.0))
        w1 = wpost_ref[:D, :]
        w2 = wpost_ref[D:, :]
        out_ref[...] = (
            jax.lax.dot_general(gsum, w1, (((1,), (0,)), ((), ())),
                                preferred_element_type=jnp.float32)
            + jax.lax.dot_general(gmean, w2, (((1,), (0,)), ((), ())),
                                  preferred_element_type=jnp.float32)
            + bpost_ref[...])


def kernel(x, event, W_pre, b_pre, W_post, b_post):
    ev = event.astype(jnp.int32)
    ev2 = ev.reshape(NB, R)
    first = ev2[:, 0]
    last = ev2[:, -1]
    base = first - (first % 8)
    nw = (last - base) // W + 1
    off = jnp.concatenate(
        [jnp.zeros((1,), jnp.int32), jnp.cumsum(nw).astype(jnp.int32)])
    total = off[NB]
    idx = jnp.arange(G, dtype=jnp.int32)
    t = jnp.clip(jnp.searchsorted(off, idx, side='right') - 1, 0, NB - 1
                 ).astype(jnp.int32)
    k = idx - off[t]
    ws = base[t] + k * W
    valid = (idx < total).astype(jnp.int32)
    ws = jnp.where(valid == 1, ws, 0)
    sched = jnp.stack([t, ws, valid]).astype(jnp.int32)

    grid_spec = pltpu.PrefetchScalarGridSpec(
        num_scalar_prefetch=1,
        grid=(G,),
        in_specs=[
            pl.BlockSpec((1, 1, R), lambda i, s: (s[0, i], 0, 0)),
            pl.BlockSpec((1, R, D), lambda i, s: (s[0, i], 0, 0)),
            pl.BlockSpec((D, D), lambda i, s: (0, 0)),
            pl.BlockSpec((1, D), lambda i, s: (0, 0)),
            pl.BlockSpec((2 * D, D), lambda i, s: (0, 0)),
            pl.BlockSpec((1, D), lambda i, s: (0, 0)),
        ],
        out_specs=pl.BlockSpec((E, D), lambda i, s: (0, 0)),
        scratch_shapes=[
            pltpu.VMEM((EPAD, D), jnp.float32),
            pltpu.VMEM((EPAD, 1), jnp.float32),
            pltpu.VMEM((R, 2 * D), jnp.bfloat16),
        ],
    )
    return pl.pallas_call(
        _body,
        grid_spec=grid_spec,
        out_shape=jax.ShapeDtypeStruct((E, D), jnp.float32),
    )(sched, ev.reshape(NB, 1, R), x.reshape(NB, R, D), W_pre,
      b_pre.reshape(1, D), W_post, b_post.reshape(1, D))


# speedup vs baseline: 11.6271x; 1.0085x over previous
import jax
import jax.numpy as jnp
from jax.experimental import pallas as pl
from jax.experimental.pallas import tpu as pltpu

N_NODES = 100000
D = 128
E = 1024
R = 10000
NB = N_NODES // R
W = 128
G = 20
EPAD = E + W


def _body(s_ref, ev_ref, x_ref, wpre_ref, bpre_ref, wpost_ref, bpost_ref,
          out_ref, gsum_ref, cnt_ref, ftx_ref):
    i = pl.program_id(0)
    t = s_ref[0, i]
    ws = s_ref[1, i]
    valid = s_ref[2, i] == 1
    prev_t = jnp.where(i == 0, -1, s_ref[0, jnp.maximum(i - 1, 0)])

    @pl.when(i == 0)
    def _init():
        gsum_ref[...] = jnp.zeros_like(gsum_ref)
        cnt_ref[...] = jnp.zeros_like(cnt_ref)
        ftx_ref[:, D:] = jnp.ones((R, D), jnp.bfloat16)

    @pl.when(t != prev_t)
    def _new_tile():
        xb = x_ref[0].astype(jnp.bfloat16)
        wb = wpre_ref[...].astype(jnp.bfloat16)
        pre = jax.lax.dot_general(xb, wb, (((1,), (0,)), ((), ())),
                                  preferred_element_type=jnp.float32)
        ftx_ref[:, :D] = jnp.maximum(pre + bpre_ref[...], 0.0
                                     ).astype(jnp.bfloat16)

    @pl.when(valid)
    def _accum():
        ev_rel = (ev_ref[0, 0, :] - ws).astype(jnp.int16)
        iota = jax.lax.broadcasted_iota(jnp.int16, (W, R), 0)
        ohb = (iota == jnp.broadcast_to(ev_rel[None, :], (W, R))
               ).astype(jnp.bfloat16)
        h = R // 4
        parts = [jax.lax.dot_general(ohb[:, j * h:(j + 1) * h],
                                     ftx_ref[j * h:(j + 1) * h, :],
                                     (((1,), (0,)), ((), ())),
                                     preferred_element_type=jnp.float32)
                 for j in range(4)]
        res = (parts[0] + parts[1]) + (parts[2] + parts[3])
        gsum_ref[pl.ds(ws, W), :] += res[:, :D]
        cnt_ref[pl.ds(ws, W), :] += res[:, D:D + 1]

    @pl.when(i == G - 1)
    def _finish():
        gsum = gsum_ref[:E, :]
        gmean = gsum * (1.0 / jnp.maximum(cnt_ref[:E, :], 1.0))
        w1 = wpost_ref[:D, :]
        w2 = wpost_ref[D:, :]
        out_ref[...] = (
            jax.lax.dot_general(gsum, w1, (((1,), (0,)), ((), ())),
                                preferred_element_type=jnp.float32)
            + jax.lax.dot_general(gmean, w2, (((1,), (0,)), ((), ())),
                                  preferred_element_type=jnp.float32)
            + bpost_ref[...])


def kernel(x, event, W_pre, b_pre, W_post, b_post):
    ev = event.astype(jnp.int32)
    ev2 = ev.reshape(NB, R)
    first = ev2[:, 0]
    last = ev2[:, -1]
    base = first - (first % 8)
    nw = (last - base) // W + 1
    off = jnp.concatenate(
        [jnp.zeros((1,), jnp.int32), jnp.cumsum(nw).astype(jnp.int32)])
    total = off[NB]
    idx = jnp.arange(G, dtype=jnp.int32)
    t = jnp.clip(jnp.searchsorted(off, idx, side='right') - 1, 0, NB - 1
                 ).astype(jnp.int32)
    k = idx - off[t]
    ws = base[t] + k * W
    valid = (idx < total).astype(jnp.int32)
    ws = jnp.where(valid == 1, ws, 0)
    sched = jnp.stack([t, ws, valid]).astype(jnp.int32)

    grid_spec = pltpu.PrefetchScalarGridSpec(
        num_scalar_prefetch=1,
        grid=(G,),
        in_specs=[
            pl.BlockSpec((1, 1, R), lambda i, s: (s[0, i], 0, 0)),
            pl.BlockSpec((1, R, D), lambda i, s: (s[0, i], 0, 0)),
            pl.BlockSpec((D, D), lambda i, s: (0, 0)),
            pl.BlockSpec((1, D), lambda i, s: (0, 0)),
            pl.BlockSpec((2 * D, D), lambda i, s: (0, 0)),
            pl.BlockSpec((1, D), lambda i, s: (0, 0)),
        ],
        out_specs=pl.BlockSpec((E, D), lambda i, s: (0, 0)),
        scratch_shapes=[
            pltpu.VMEM((EPAD, D), jnp.float32),
            pltpu.VMEM((EPAD, 1), jnp.float32),
            pltpu.VMEM((R, 2 * D), jnp.bfloat16),
        ],
    )
    return pl.pallas_call(
        _body,
        grid_spec=grid_spec,
        out_shape=jax.ShapeDtypeStruct((E, D), jnp.float32),
    )(sched, ev.reshape(NB, 1, R), x.reshape(NB, R, D), W_pre,
      b_pre.reshape(1, D), W_post, b_post.reshape(1, D))


# speedup vs baseline: 14.0011x; 1.2042x over previous
import jax
import jax.numpy as jnp
from jax.experimental import pallas as pl
from jax.experimental.pallas import tpu as pltpu

N_NODES = 100000
D = 128
E = 1024
R = 10000
NB = N_NODES // R
W = 128
EPAD = E + W


def _body(ev_ref, x_ref, wpre_ref, bpre_ref, wpost_ref, bpost_ref,
          out_ref, gsum_ref, cnt_ref, ftx_ref):
    i = pl.program_id(0)

    @pl.when(i == 0)
    def _init():
        gsum_ref[...] = jnp.zeros_like(gsum_ref)
        cnt_ref[...] = jnp.zeros_like(cnt_ref)
        ftx_ref[:, D:] = jnp.ones((R, D), jnp.bfloat16)

    xb = x_ref[0].astype(jnp.bfloat16)
    wb = wpre_ref[...].astype(jnp.bfloat16)
    pre = jax.lax.dot_general(xb, wb, (((1,), (0,)), ((), ())),
                              preferred_element_type=jnp.float32)
    ftx_ref[:, :D] = jnp.maximum(pre + bpre_ref[...], 0.0).astype(jnp.bfloat16)

    first = ev_ref[0, 0, 0]
    last = ev_ref[0, 0, R - 1]
    base = first - first % 8
    n_win = (last - base) // W + 1
    ev16 = ev_ref[0, 0, :].astype(jnp.int16)

    def _one_window(w, _):
        ws = base + w * W
        ev_rel = ev16 - ws.astype(jnp.int16)
        iota = jax.lax.broadcasted_iota(jnp.int16, (W, R), 0)
        ohb = (iota == jnp.broadcast_to(ev_rel[None, :], (W, R))
               ).astype(jnp.bfloat16)
        h = R // 4
        parts = [jax.lax.dot_general(ohb[:, j * h:(j + 1) * h],
                                     ftx_ref[j * h:(j + 1) * h, :],
                                     (((1,), (0,)), ((), ())),
                                     preferred_element_type=jnp.float32)
                 for j in range(4)]
        res = (parts[0] + parts[1]) + (parts[2] + parts[3])
        gsum_ref[pl.ds(ws, W), :] += res[:, :D]
        cnt_ref[pl.ds(ws, W), :] += res[:, D:D + 1]
        return 0

    jax.lax.fori_loop(0, n_win, _one_window, 0)

    @pl.when(i == NB - 1)
    def _finish():
        gsum = gsum_ref[:E, :]
        gmean = gsum * (1.0 / jnp.maximum(cnt_ref[:E, :], 1.0))
        w1 = wpost_ref[:D, :]
        w2 = wpost_ref[D:, :]
        out_ref[...] = (
            jax.lax.dot_general(gsum, w1, (((1,), (0,)), ((), ())),
                                preferred_element_type=jnp.float32)
            + jax.lax.dot_general(gmean, w2, (((1,), (0,)), ((), ())),
                                  preferred_element_type=jnp.float32)
            + bpost_ref[...])


def kernel(x, event, W_pre, b_pre, W_post, b_post):
    ev = event.astype(jnp.int32)
    return pl.pallas_call(
        _body,
        grid=(NB,),
        in_specs=[
            pl.BlockSpec((1, 1, R), lambda i: (i, 0, 0)),
            pl.BlockSpec((1, R, D), lambda i: (i, 0, 0)),
            pl.BlockSpec((D, D), lambda i: (0, 0)),
            pl.BlockSpec((1, D), lambda i: (0, 0)),
            pl.BlockSpec((2 * D, D), lambda i: (0, 0)),
            pl.BlockSpec((1, D), lambda i: (0, 0)),
        ],
        out_specs=pl.BlockSpec((E, D), lambda i: (0, 0)),
        out_shape=jax.ShapeDtypeStruct((E, D), jnp.float32),
        scratch_shapes=[
            pltpu.VMEM((EPAD, D), jnp.float32),
            pltpu.VMEM((EPAD, 1), jnp.float32),
            pltpu.VMEM((R, 2 * D), jnp.bfloat16),
        ],
    )(ev.reshape(NB, 1, R), x.reshape(NB, R, D), W_pre,
      b_pre.reshape(1, D), W_post, b_post.reshape(1, D))
